# Initial kernel scaffold; baseline (speedup 1.0000x reference)
import jax
import jax.numpy as jnp
from jax.experimental import pallas as pl


def kernel(seq1, seq2, adj, W_gcn, b_gcn, W_disc, b_disc):
    raise NotImplementedError("write your pallas kernel here")


# speedup vs baseline: 4.7147x; 4.7147x over previous
import functools

import jax
import jax.numpy as jnp
from jax import lax
from jax.experimental import pallas as pl
from jax.experimental.pallas import tpu as pltpu
from jax.experimental.pallas import tpu_sc as plsc

N = 10000
D = 128
E = 320000

NT = 16
CH = 128
IB = 32
NB = 5
CPT = IB * NB
E_PAD = NT * CPT * CH
N_PAD = 10240
RPT = N_PAD // NT
DUMMY = N_PAD - 8

MM_BLK = 400
PB_BLK = 512


def _matmul_pair(seq1, seq2, w, b):
  def body(s1, s2, w_ref, b_ref, o1, o2):
    wv = w_ref[...]
    bv = b_ref[...]
    o1[...] = jnp.dot(s1[...], wv, preferred_element_type=jnp.float32) + bv
    o2[...] = jnp.dot(s2[...], wv, preferred_element_type=jnp.float32) + bv

  grid = N // MM_BLK
  return pl.pallas_call(
      body,
      grid=(grid,),
      in_specs=[
          pl.BlockSpec((MM_BLK, D), lambda i: (i, 0)),
          pl.BlockSpec((MM_BLK, D), lambda i: (i, 0)),
          pl.BlockSpec((D, D), lambda i: (0, 0)),
          pl.BlockSpec((1, D), lambda i: (0, 0)),
      ],
      out_specs=[
          pl.BlockSpec((MM_BLK, D), lambda i: (i, 0)),
          pl.BlockSpec((MM_BLK, D), lambda i: (i, 0)),
      ],
      out_shape=[
          jax.ShapeDtypeStruct((N, D), jnp.float32),
          jax.ShapeDtypeStruct((N, D), jnp.float32),
      ],
  )(seq1, seq2, w, b.reshape(1, D))


def _sc_body(pre1, pre2, srch, dsth, out1, out2,
             src_v, dst_v, rows0, rows1, sem0, sem1, acc):
  s = lax.axis_index("s")
  c = lax.axis_index("c")

  zv = jnp.zeros((16,), jnp.float32)
  @pl.loop(0, CH)
  def _zero_rows(r):
    for k in range(D // 16):
      rows0[r, pl.ds(k * 16, 16)] = zv
  for k in range(RPT // CH):
    pltpu.sync_copy(rows0, acc.at[pl.ds(s * RPT + k * CH, CH)])
  plsc.subcore_barrier()

  def run(pre, out):
    @pl.loop(0, NB)
    def _blocks(ib):
      pltpu.sync_copy(srch.at[s, pl.ds(ib * IB, IB)], src_v)
      pltpu.sync_copy(dsth.at[s, pl.ds(ib * IB, IB)], dst_v)

      pltpu.async_copy(pre.at[src_v.at[0]], rows0, sem0)

      @pl.loop(0, IB, step=2)
      def _chunks(j):
        pltpu.make_async_copy(pre.at[src_v.at[j]], rows0, sem0).wait()
        pltpu.async_copy(pre.at[src_v.at[j + 1]], rows1, sem1)
        pltpu.sync_copy(rows0, acc.at[dst_v.at[j]], add=True)
        pltpu.make_async_copy(pre.at[src_v.at[j]], rows1, sem1).wait()
        @pl.when(j + 2 < IB)
        def _():
          pltpu.async_copy(pre.at[src_v.at[j + 2]], rows0, sem0)
        pltpu.sync_copy(rows1, acc.at[dst_v.at[j + 1]], add=True)

    plsc.subcore_barrier()
    for k in range(RPT // CH):
      pltpu.sync_copy(acc.at[pl.ds(s * RPT + k * CH, CH)], rows0)
      pltpu.sync_copy(rows0, out.at[pl.ds(s * RPT + k * CH, CH)])

  @pl.when(c == 0)
  def _():
    run(pre1, out1)

  @pl.when(c == 1)
  def _():
    run(pre2, out2)


def _sc_segsum(pre1, pre2, src_p, dst_p):
  mesh = plsc.VectorSubcoreMesh(core_axis_name="c", subcore_axis_name="s")
  return pl.kernel(
      _sc_body,
      out_type=[
          jax.ShapeDtypeStruct((N_PAD, D), jnp.float32),
          jax.ShapeDtypeStruct((N_PAD, D), jnp.float32),
      ],
      mesh=mesh,
      scratch_types=[
          pltpu.VMEM((IB, CH), jnp.int32),
          pltpu.VMEM((IB, CH), jnp.int32),
          pltpu.VMEM((CH, D), jnp.float32),
          pltpu.VMEM((CH, D), jnp.float32),
          pltpu.SemaphoreType.DMA,
          pltpu.SemaphoreType.DMA,
          pltpu.VMEM_SHARED((N_PAD, D), jnp.float32),
      ],
  )(pre1, pre2, src_p, dst_p)


def _summary_wc(agg1, w_disc):
  grid = N_PAD // PB_BLK

  def body(a1, wd, wc_out, acc):
    i = pl.program_id(0)

    @pl.when(i == 0)
    def _():
      acc[...] = jnp.zeros_like(acc)

    h = a1[...]
    h = jnp.where(h > 0, h, 0.25 * h)
    rid = i * PB_BLK + lax.broadcasted_iota(jnp.int32, (PB_BLK, D), 0)
    h = jnp.where(rid < N, h, 0.0)
    acc[...] += jnp.sum(h, axis=0, keepdims=True)

    @pl.when(i == grid - 1)
    def _():
      cs = acc[...] / jnp.float32(N)
      cs = 1.0 / (1.0 + jnp.exp(-cs))
      wc = jnp.dot(wd[...], cs.reshape(D, 1), preferred_element_type=jnp.float32)
      wc_out[...] = wc.reshape(1, D)

  return pl.pallas_call(
      body,
      grid=(grid,),
      in_specs=[
          pl.BlockSpec((PB_BLK, D), lambda i: (i, 0)),
          pl.BlockSpec((D, D), lambda i: (0, 0)),
      ],
      out_specs=pl.BlockSpec((1, D), lambda i: (0, 0)),
      out_shape=jax.ShapeDtypeStruct((1, D), jnp.float32),
      scratch_shapes=[pltpu.VMEM((1, D), jnp.float32)],
  )(agg1, w_disc)


def _scores(agg1, agg2, wc, b_disc):
  grid = N_PAD // PB_BLK

  def body(a1, a2, wc_ref, b_ref, o1, o2):
    w = wc_ref[...]
    bv = b_ref[0, 0]

    def sc(a):
      h = a[...]
      h = jnp.where(h > 0, h, 0.25 * h)
      return jnp.sum(h * w, axis=1).reshape(1, 1, PB_BLK) + bv

    o1[...] = sc(a1)
    o2[...] = sc(a2)

  return pl.pallas_call(
      body,
      grid=(grid,),
      in_specs=[
          pl.BlockSpec((PB_BLK, D), lambda i: (i, 0)),
          pl.BlockSpec((PB_BLK, D), lambda i: (i, 0)),
          pl.BlockSpec((1, D), lambda i: (0, 0)),
          pl.BlockSpec((1, 1), lambda i: (0, 0)),
      ],
      out_specs=[
          pl.BlockSpec((1, 1, PB_BLK), lambda i: (i, 0, 0)),
          pl.BlockSpec((1, 1, PB_BLK), lambda i: (i, 0, 0)),
      ],
      out_shape=[
          jax.ShapeDtypeStruct((grid, 1, PB_BLK), jnp.float32),
          jax.ShapeDtypeStruct((grid, 1, PB_BLK), jnp.float32),
      ],
  )(agg1, agg2, wc, b_disc.reshape(1, 1))


def kernel(seq1, seq2, adj, W_gcn, b_gcn, W_disc, b_disc):
  src = adj[0]
  dst = adj[1]
  pad = E_PAD - E
  src_p = jnp.concatenate([src, jnp.zeros((pad,), jnp.int32)]).reshape(NT, CPT, CH)
  dst_p = jnp.concatenate([dst, jnp.full((pad,), DUMMY, jnp.int32)]).reshape(NT, CPT, CH)

  pre1, pre2 = _matmul_pair(seq1, seq2, W_gcn, b_gcn)
  agg1, agg2 = _sc_segsum(pre1, pre2, src_p, dst_p)
  wc = _summary_wc(agg1, W_disc)
  s1, s2 = _scores(agg1, agg2, wc, b_disc)
  return jnp.concatenate([s1.reshape(-1)[:N], s2.reshape(-1)[:N]], axis=0)
